# baseline (speedup 1.0000x reference)
import jax
import jax.numpy as jnp
import numpy as np
from jax.experimental import pallas as pl
from jax.experimental.pallas import tpu as pltpu

_MULT_SIZE = 1.2


def _pad_body(species_any, natoms_ref, batch_any, coordsT_any,
              neg1_any, sys16_any, zeros3_any, species_vmem,
              species_out_any, natoms_out_ref, batch_out_any, coordsT_out_any,
              true_atoms_ref,
              s0, s1, s2, s3, s4, s5):
    nat = species_vmem.shape[0]
    pad_nat = true_atoms_ref.shape[0]
    add = pad_nat - nat
    nsys = natoms_ref.shape[0]

    cps = [
        pltpu.make_async_copy(species_any, species_out_any.at[pl.ds(0, nat)], s0),
        pltpu.make_async_copy(neg1_any, species_out_any.at[pl.ds(nat, add)], s1),
        pltpu.make_async_copy(batch_any, batch_out_any.at[pl.ds(0, nat)], s2),
        pltpu.make_async_copy(sys16_any, batch_out_any.at[pl.ds(nat, add)], s3),
        pltpu.make_async_copy(coordsT_any, coordsT_out_any.at[:, pl.ds(0, nat)], s4),
        pltpu.make_async_copy(zeros3_any, coordsT_out_any.at[:, pl.ds(nat, add)], s5),
    ]
    for c in cps:
        c.start()

    s = species_vmem[...]
    true_atoms_ref[0:nat] = s > 0
    true_atoms_ref[nat:pad_nat] = jnp.zeros((add,), jnp.bool_)
    natoms_out_ref[0:nsys] = natoms_ref[...]
    natoms_out_ref[nsys:nsys + 1] = jnp.full((1,), add, natoms_ref.dtype)

    for c in cps:
        c.wait()


def kernel(species, natoms, batch_index, coordinates, cells):
    nat = species.shape[0]
    nsys = natoms.shape[0]
    pad_nat = int(_MULT_SIZE * nat) + 1
    add = pad_nat - nat
    ndim = coordinates.shape[1]

    neg1 = jnp.asarray(np.full((add,), -1, species.dtype))
    sys16 = jnp.asarray(np.full((add,), nsys, batch_index.dtype))
    zeros3 = jnp.asarray(np.zeros((ndim, add), coordinates.dtype))

    out_shape = (
        jax.ShapeDtypeStruct((pad_nat,), species.dtype),
        jax.ShapeDtypeStruct((nsys + 1,), natoms.dtype),
        jax.ShapeDtypeStruct((pad_nat,), batch_index.dtype),
        jax.ShapeDtypeStruct((ndim, pad_nat), coordinates.dtype),
        jax.ShapeDtypeStruct((pad_nat,), jnp.bool_),
    )
    any_spec = pl.BlockSpec(memory_space=pl.ANY)
    grid_spec = pltpu.PrefetchScalarGridSpec(
        num_scalar_prefetch=0,
        in_specs=[any_spec, pl.BlockSpec(memory_space=pltpu.MemorySpace.VMEM), any_spec,
                  any_spec, any_spec, any_spec, any_spec,
                  pl.BlockSpec(memory_space=pltpu.MemorySpace.VMEM)],
        out_specs=[any_spec, pl.BlockSpec(memory_space=pltpu.MemorySpace.VMEM), any_spec,
                   any_spec, pl.BlockSpec(memory_space=pltpu.MemorySpace.VMEM)],
        scratch_shapes=[pltpu.SemaphoreType.DMA] * 6,
    )
    (species_out, natoms_out, batch_out, coordsT_out,
     true_atoms) = pl.pallas_call(_pad_body, grid_spec=grid_spec,
                                  out_shape=out_shape)(
        species, natoms, batch_index, coordinates.T, neg1, sys16, zeros3,
        species)

    cells_out = jnp.concatenate(
        [cells, jnp.eye(cells.shape[1], dtype=cells.dtype)[None, :, :]], axis=0)
    true_sys = jnp.arange(nsys + 1) < nsys
    return (species_out, natoms_out, batch_out, coordsT_out.T, cells_out,
            true_atoms, true_sys)


# speedup vs baseline: 5.8542x; 5.8542x over previous
import jax
import jax.numpy as jnp
from jax.experimental import pallas as pl

_MULT_SIZE = 1.2


def _pad_body(species_ref, natoms_ref, batch_ref, coordsT_ref,
              species_out_ref, natoms_out_ref, batch_out_ref, coordsT_out_ref,
              true_atoms_ref):
    nat = species_ref.shape[0]
    nsys = natoms_ref.shape[0]
    pad_nat = species_out_ref.shape[0]
    add = pad_nat - nat

    s = species_ref[...]
    species_out_ref[0:nat] = s
    species_out_ref[nat:pad_nat] = jnp.full((add,), -1, species_ref.dtype)
    true_atoms_ref[0:nat] = s > 0
    true_atoms_ref[nat:pad_nat] = jnp.zeros((add,), jnp.bool_)

    batch_out_ref[0:nat] = batch_ref[...]
    batch_out_ref[nat:pad_nat] = jnp.full((add,), nsys, batch_ref.dtype)

    coordsT_out_ref[:, 0:nat] = coordsT_ref[...]
    coordsT_out_ref[:, nat:pad_nat] = jnp.zeros(
        (coordsT_ref.shape[0], add), coordsT_ref.dtype)

    natoms_out_ref[0:nsys] = natoms_ref[...]
    natoms_out_ref[nsys:nsys + 1] = jnp.full((1,), add, natoms_ref.dtype)


def kernel(species, natoms, batch_index, coordinates, cells):
    nat = species.shape[0]
    nsys = natoms.shape[0]
    pad_nat = int(_MULT_SIZE * nat) + 1
    ndim = coordinates.shape[1]

    out_shape = (
        jax.ShapeDtypeStruct((pad_nat,), species.dtype),
        jax.ShapeDtypeStruct((nsys + 1,), natoms.dtype),
        jax.ShapeDtypeStruct((pad_nat,), batch_index.dtype),
        jax.ShapeDtypeStruct((ndim, pad_nat), coordinates.dtype),
        jax.ShapeDtypeStruct((pad_nat,), jnp.bool_),
    )
    (species_out, natoms_out, batch_out, coordsT_out,
     true_atoms) = pl.pallas_call(_pad_body, out_shape=out_shape)(
        species, natoms, batch_index, coordinates.T)

    cells_out = jnp.concatenate(
        [cells, jnp.eye(cells.shape[1], dtype=cells.dtype)[None, :, :]], axis=0)
    true_sys = jnp.arange(nsys + 1) < nsys
    return (species_out, natoms_out, batch_out, coordsT_out.T, cells_out,
            true_atoms, true_sys)
